# baseline (speedup 1.0000x reference)
import functools

import jax
import jax.numpy as jnp
from jax import lax
from jax.experimental import pallas as pl
from jax.experimental.pallas import tpu as pltpu
from jax.experimental.pallas import tpu_sc as plsc

_N = 10000
_E = 320000
_HID = 64
_MU2P = 0.1
_BN_EPS = 1e-5
_NC = 2
_NS = 16
_NW = _NC * _NS
_CH = 128
_CHUNKS = 80
_EPW = _CHUNKS * _CH
_NP = 10112
_RPT = _NP // _NS
_DEGW = 16


def _sc_mesh():
    return plsc.VectorSubcoreMesh(core_axis_name="c", subcore_axis_name="s")


_SC_PARAMS = pltpu.CompilerParams(use_tc_tiling_on_sc=False)


def _deg_partials(rowp, ones, zeros):
    @functools.partial(
        pl.kernel,
        out_type=jax.ShapeDtypeStruct((_NC, _NP, _DEGW), jnp.float32),
        mesh=_sc_mesh(),
        scratch_types=[
            pltpu.VMEM((_CHUNKS, _CH), jnp.int32),
            pltpu.VMEM((_CH, _DEGW), jnp.float32),
            pltpu.VMEM_SHARED((_NP, _DEGW), jnp.float32),
        ],
        compiler_params=_SC_PARAMS,
    )
    def k(row_hbm, ones_hbm, zero_hbm, out_hbm, idx_v, ones_v, acc_sh):
        c = lax.axis_index("c")
        s = lax.axis_index("s")
        wid = c * jnp.int32(_NS) + s
        sl = pl.ds(s * jnp.int32(_RPT), _RPT)
        pltpu.sync_copy(zero_hbm.at[sl], acc_sh.at[sl])
        pltpu.sync_copy(row_hbm.at[wid], idx_v)
        pltpu.sync_copy(ones_hbm, ones_v)
        plsc.subcore_barrier()

        @pl.loop(0, _CHUNKS)
        def _(g):
            pltpu.sync_copy(ones_v, acc_sh.at[idx_v.at[g]], add=True)

        plsc.subcore_barrier()
        pltpu.sync_copy(acc_sh.at[sl], out_hbm.at[c, sl])

    return k(rowp, ones, zeros)


def _spmv_partials(u, rowp, colp, zeros):
    @functools.partial(
        pl.kernel,
        out_type=jax.ShapeDtypeStruct((_NC, _NP, _HID), jnp.float32),
        mesh=_sc_mesh(),
        scratch_types=[
            pltpu.VMEM((_CHUNKS, _CH), jnp.int32),
            pltpu.VMEM((_CHUNKS, _CH), jnp.int32),
            pltpu.VMEM((4, _CH, _HID), jnp.float32),
            pltpu.VMEM_SHARED((_NP, _HID), jnp.float32),
            pltpu.SemaphoreType.DMA((4,)),
            pltpu.SemaphoreType.DMA((4,)),
        ],
        compiler_params=_SC_PARAMS,
    )
    def k(u_hbm, row_hbm, col_hbm, zero_hbm, out_hbm, row_v, col_v, rows_v,
          acc_sh, gsem, ssem):
        c = lax.axis_index("c")
        s = lax.axis_index("s")
        wid = c * jnp.int32(_NS) + s
        sl = pl.ds(s * jnp.int32(_RPT), _RPT)
        pltpu.sync_copy(zero_hbm.at[sl], acc_sh.at[sl])
        pltpu.sync_copy(row_hbm.at[wid], row_v)
        pltpu.sync_copy(col_hbm.at[wid], col_v)
        plsc.subcore_barrier()

        def gather_start(i, b):
            pltpu.async_copy(u_hbm.at[col_v.at[i]], rows_v.at[b], gsem.at[b])

        def gather_wait(b):
            pltpu.make_async_copy(
                u_hbm.at[col_v.at[0]], rows_v.at[b], gsem.at[b]).wait()

        def scatter_start(i, b):
            pltpu.async_copy(rows_v.at[b], acc_sh.at[row_v.at[i]], ssem.at[b],
                             add=True)

        def scatter_wait(b):
            pltpu.make_async_copy(
                rows_v.at[b], acc_sh.at[row_v.at[0]], ssem.at[b]).wait()

        gather_start(0, 0)
        gather_start(1, 1)

        @pl.loop(0, _CHUNKS, step=4)
        def _(g):
            for b in range(4):
                i = g + jnp.int32(b)
                b2 = (b + 2) % 4
                gather_wait(b)
                scatter_start(i, b)

                @pl.when(i >= 2)
                def _():
                    scatter_wait(b2)

                @pl.when(i + 2 < _CHUNKS)
                def _():
                    gather_start(i + 2, b2)

        scatter_wait(2)
        scatter_wait(3)
        plsc.subcore_barrier()
        pltpu.sync_copy(acc_sh.at[sl], out_hbm.at[c, sl])

    return k(u, rowp, colp, zeros)


def _prep_h(xp, W1, b1, gamma, beta, mean, var):
    def body(x_ref, w_ref, b_ref, g_ref, be_ref, m_ref, v_ref, o_ref):
        hm = jnp.dot(x_ref[...], w_ref[...], preferred_element_type=jnp.float32)
        inv = g_ref[...] * lax.rsqrt(v_ref[...] + _BN_EPS)
        o_ref[...] = jnp.maximum((hm + b_ref[...] - m_ref[...]) * inv + be_ref[...], 0.0)

    return pl.pallas_call(
        body, out_shape=jax.ShapeDtypeStruct((_NP, _HID), jnp.float32)
    )(xp, W1, b1, gamma, beta, mean, var)


def _node_coeffs(d_ref):
    deg = d_ref[0, :, 0:1] + d_ref[1, :, 0:1]
    pos = deg > 0.0
    dis = jnp.where(pos, lax.rsqrt(deg), 0.0)
    alpha = 1.0 / (jnp.where(pos, 1.0, 0.0) + _MU2P)
    return dis, alpha


def _make_u0(degp, h):
    def body(d_ref, h_ref, o_ref):
        dis, _ = _node_coeffs(d_ref)
        o_ref[...] = dis * h_ref[...]

    return pl.pallas_call(
        body, out_shape=jax.ShapeDtypeStruct((_NP, _HID), jnp.float32)
    )(degp, h)


def _combine_u(degp, s, h):
    def body(d_ref, s_ref, h_ref, o_ref):
        dis, alpha = _node_coeffs(d_ref)
        f = alpha * (dis * (s_ref[0] + s_ref[1])) + (_MU2P * alpha) * h_ref[...]
        o_ref[...] = dis * f

    return pl.pallas_call(
        body, out_shape=jax.ShapeDtypeStruct((_NP, _HID), jnp.float32)
    )(degp, s, h)


def _final(degp, s, h, W2, b2):
    def body(d_ref, s_ref, h_ref, w_ref, b_ref, o_ref):
        dis, alpha = _node_coeffs(d_ref)
        f = alpha * (dis * (s_ref[0] + s_ref[1])) + (_MU2P * alpha) * h_ref[...]
        o = jnp.dot(f, w_ref[...], preferred_element_type=jnp.float32) + b_ref[...]
        m = jnp.max(o, axis=1, keepdims=True)
        y = o - m
        o_ref[...] = y - jnp.log(jnp.sum(jnp.exp(y), axis=1, keepdims=True))

    return pl.pallas_call(
        body, out_shape=jax.ShapeDtypeStruct((_NP, _HID), jnp.float32)
    )(degp, s, h, W2, b2)


def kernel(x, edge_index, W1, b1, gamma, beta, running_mean, running_var, W2, b2):
    with jax.enable_x64(False):
        return _kernel32(x, edge_index, W1, b1, gamma, beta, running_mean,
                         running_var, W2, b2)


def _kernel32(x, edge_index, W1, b1, gamma, beta, running_mean, running_var, W2, b2):
    x = x.astype(jnp.float32)
    row = edge_index[0].astype(jnp.int32)
    col = edge_index[1].astype(jnp.int32)
    pad = _NW * _EPW - _E
    rowp = jnp.concatenate([row, jnp.full((pad,), _N, jnp.int32)]).reshape(
        _NW, _CHUNKS, _CH)
    colp = jnp.concatenate([col, jnp.full((pad,), _N, jnp.int32)]).reshape(
        _NW, _CHUNKS, _CH)
    xp = jnp.pad(x, ((0, _NP - _N), (0, 0)))
    ones16 = jnp.ones((_CH, _DEGW), jnp.float32)
    z16 = jnp.zeros((_NP, _DEGW), jnp.float32)
    z64 = jnp.zeros((_NP, _HID), jnp.float32)

    h = _prep_h(xp, W1.astype(jnp.float32), b1, gamma, beta,
                running_mean, running_var)
    degp = _deg_partials(rowp, ones16, z16)
    u = _make_u0(degp, h)
    s = _spmv_partials(u, rowp, colp, z64)
    u = _combine_u(degp, s, h)
    s = _spmv_partials(u, rowp, colp, z64)
    out = _final(degp, s, h, W2.astype(jnp.float32), b2)
    return out[:_N]


# speedup vs baseline: 2.1150x; 2.1150x over previous
import functools

import jax
import jax.numpy as jnp
from jax import lax
from jax.experimental import pallas as pl
from jax.experimental.pallas import tpu as pltpu
from jax.experimental.pallas import tpu_sc as plsc

_N = 10000
_E = 320000
_HID = 64
_HH = 32
_MU2P = 0.1
_BN_EPS = 1e-5
_NC = 2
_NS = 16
_NW = _NC * _NS
_CH = 128
_CHUNKS = 80
_EPW = _CHUNKS * _CH
_NP = 10112
_RPT = _NP // _NS
_DEGW = 16


def _sc_mesh():
    return plsc.VectorSubcoreMesh(core_axis_name="c", subcore_axis_name="s")


_SC_PARAMS = pltpu.CompilerParams(use_tc_tiling_on_sc=False)


def _deg_partials(rowp, ones, zeros):
    @functools.partial(
        pl.kernel,
        out_type=jax.ShapeDtypeStruct((_NC, _NP, _DEGW), jnp.float32),
        mesh=_sc_mesh(),
        scratch_types=[
            pltpu.VMEM((_CHUNKS, _CH), jnp.int32),
            pltpu.VMEM((_CH, _DEGW), jnp.float32),
            pltpu.VMEM_SHARED((_NP, _DEGW), jnp.float32),
        ],
        compiler_params=_SC_PARAMS,
    )
    def k(row_hbm, ones_hbm, zero_hbm, out_hbm, idx_v, ones_v, acc_sh):
        c = lax.axis_index("c")
        s = lax.axis_index("s")
        wid = c * jnp.int32(_NS) + s
        sl = pl.ds(s * jnp.int32(_RPT), _RPT)
        pltpu.sync_copy(zero_hbm.at[sl], acc_sh.at[sl])
        pltpu.sync_copy(row_hbm.at[wid], idx_v)
        pltpu.sync_copy(ones_hbm, ones_v)
        plsc.subcore_barrier()

        @pl.loop(0, _CHUNKS)
        def _(g):
            pltpu.sync_copy(ones_v, acc_sh.at[idx_v.at[g]], add=True)

        plsc.subcore_barrier()
        pltpu.sync_copy(acc_sh.at[sl], out_hbm.at[c, sl])

    return k(rowp, ones, zeros)


def _spmv_partials(u, rowp, colp, zeros):
    @functools.partial(
        pl.kernel,
        out_type=jax.ShapeDtypeStruct((_NC, _NP, _HID), jnp.float32),
        mesh=_sc_mesh(),
        scratch_types=[
            pltpu.VMEM((_CHUNKS, _CH), jnp.int32),
            pltpu.VMEM((_CHUNKS, _CH), jnp.int32),
            pltpu.VMEM((4, _CH, _HH), jnp.float32),
            pltpu.VMEM_SHARED((_NP, _HH), jnp.float32),
            pltpu.VMEM_SHARED((_NP, _HH), jnp.float32),
            pltpu.SemaphoreType.DMA((4,)),
            pltpu.SemaphoreType.DMA((4,)),
        ],
        compiler_params=_SC_PARAMS,
    )
    def k(u_hbm, row_hbm, col_hbm, zero_hbm, out_hbm, row_v, col_v, rows_v,
          acc_sh, u_sh, gsem, ssem):
        c = lax.axis_index("c")
        s = lax.axis_index("s")
        wid = c * jnp.int32(_NS) + s
        sl = pl.ds(s * jnp.int32(_RPT), _RPT)
        pltpu.sync_copy(row_hbm.at[wid], row_v)
        pltpu.sync_copy(col_hbm.at[wid], col_v)

        def gather_start(i, b):
            pltpu.async_copy(u_sh.at[col_v.at[i]], rows_v.at[b], gsem.at[b])

        def gather_wait(b):
            pltpu.make_async_copy(
                u_sh.at[col_v.at[0]], rows_v.at[b], gsem.at[b]).wait()

        def scatter_start(i, b):
            pltpu.async_copy(rows_v.at[b], acc_sh.at[row_v.at[i]], ssem.at[b],
                             add=True)

        def scatter_wait(b):
            pltpu.make_async_copy(
                rows_v.at[b], acc_sh.at[row_v.at[0]], ssem.at[b]).wait()

        for h in range(2):
            hh = pl.ds(h * _HH, _HH)
            pltpu.sync_copy(zero_hbm.at[sl], acc_sh.at[sl])
            pltpu.sync_copy(u_hbm.at[sl, hh], u_sh.at[sl])
            plsc.subcore_barrier()

            gather_start(0, 0)
            gather_start(1, 1)

            @pl.loop(0, _CHUNKS, step=4)
            def _(g):
                for b in range(4):
                    i = g + jnp.int32(b)
                    b2 = (b + 2) % 4
                    gather_wait(b)
                    scatter_start(i, b)

                    @pl.when(i >= 2)
                    def _():
                        scatter_wait(b2)

                    @pl.when(i + 2 < _CHUNKS)
                    def _():
                        gather_start(i + 2, b2)

            scatter_wait(2)
            scatter_wait(3)
            plsc.subcore_barrier()
            pltpu.sync_copy(acc_sh.at[sl], out_hbm.at[c, sl, hh])

    return k(u, rowp, colp, zeros)


def _prep_h(xp, W1, b1, gamma, beta, mean, var):
    def body(x_ref, w_ref, b_ref, g_ref, be_ref, m_ref, v_ref, o_ref):
        hm = jnp.dot(x_ref[...], w_ref[...], preferred_element_type=jnp.float32)
        inv = g_ref[...] * lax.rsqrt(v_ref[...] + _BN_EPS)
        o_ref[...] = jnp.maximum((hm + b_ref[...] - m_ref[...]) * inv + be_ref[...], 0.0)

    return pl.pallas_call(
        body, out_shape=jax.ShapeDtypeStruct((_NP, _HID), jnp.float32)
    )(xp, W1, b1, gamma, beta, mean, var)


def _node_coeffs(d_ref):
    deg = d_ref[0, :, 0:1] + d_ref[1, :, 0:1]
    pos = deg > 0.0
    dis = jnp.where(pos, lax.rsqrt(deg), 0.0)
    alpha = 1.0 / (jnp.where(pos, 1.0, 0.0) + _MU2P)
    return dis, alpha


def _make_u0(degp, h):
    def body(d_ref, h_ref, o_ref):
        dis, _ = _node_coeffs(d_ref)
        o_ref[...] = dis * h_ref[...]

    return pl.pallas_call(
        body, out_shape=jax.ShapeDtypeStruct((_NP, _HID), jnp.float32)
    )(degp, h)


def _combine_u(degp, s, h):
    def body(d_ref, s_ref, h_ref, o_ref):
        dis, alpha = _node_coeffs(d_ref)
        f = alpha * (dis * (s_ref[0] + s_ref[1])) + (_MU2P * alpha) * h_ref[...]
        o_ref[...] = dis * f

    return pl.pallas_call(
        body, out_shape=jax.ShapeDtypeStruct((_NP, _HID), jnp.float32)
    )(degp, s, h)


def _final(degp, s, h, W2, b2):
    def body(d_ref, s_ref, h_ref, w_ref, b_ref, o_ref):
        dis, alpha = _node_coeffs(d_ref)
        f = alpha * (dis * (s_ref[0] + s_ref[1])) + (_MU2P * alpha) * h_ref[...]
        o = jnp.dot(f, w_ref[...], preferred_element_type=jnp.float32) + b_ref[...]
        m = jnp.max(o, axis=1, keepdims=True)
        y = o - m
        o_ref[...] = y - jnp.log(jnp.sum(jnp.exp(y), axis=1, keepdims=True))

    return pl.pallas_call(
        body, out_shape=jax.ShapeDtypeStruct((_NP, _HID), jnp.float32)
    )(degp, s, h, W2, b2)


def kernel(x, edge_index, W1, b1, gamma, beta, running_mean, running_var, W2, b2):
    with jax.enable_x64(False):
        return _kernel32(x, edge_index, W1, b1, gamma, beta, running_mean,
                         running_var, W2, b2)


def _kernel32(x, edge_index, W1, b1, gamma, beta, running_mean, running_var, W2, b2):
    x = x.astype(jnp.float32)
    row = edge_index[0].astype(jnp.int32)
    col = edge_index[1].astype(jnp.int32)
    pad = _NW * _EPW - _E
    rowp = jnp.concatenate([row, jnp.full((pad,), _N, jnp.int32)]).reshape(
        _NW, _CHUNKS, _CH)
    colp = jnp.concatenate([col, jnp.full((pad,), _N, jnp.int32)]).reshape(
        _NW, _CHUNKS, _CH)
    xp = jnp.pad(x, ((0, _NP - _N), (0, 0)))
    ones16 = jnp.ones((_CH, _DEGW), jnp.float32)
    z16 = jnp.zeros((_NP, _DEGW), jnp.float32)
    z32 = jnp.zeros((_NP, _HH), jnp.float32)

    h = _prep_h(xp, W1.astype(jnp.float32), b1, gamma, beta,
                running_mean, running_var)
    degp = _deg_partials(rowp, ones16, z16)
    u = _make_u0(degp, h)
    s = _spmv_partials(u, rowp, colp, z32)
    u = _combine_u(degp, s, h)
    s = _spmv_partials(u, rowp, colp, z32)
    out = _final(degp, s, h, W2.astype(jnp.float32), b2)
    return out[:_N]
